# Initial kernel scaffold; baseline (speedup 1.0000x reference)
import jax
import jax.numpy as jnp
from jax.experimental import pallas as pl


def kernel(sigma2_schedule, t_schedule, flat_noise_sigma_chol):
    raise NotImplementedError("write your pallas kernel here")


# speedup vs baseline: 7.9660x; 7.9660x over previous
import jax
import jax.numpy as jnp
from jax.experimental import pallas as pl
from jax.experimental.pallas import tpu as pltpu

_D = 128
_FLAT = _D * (_D + 1) // 2
_TB = 256


def _body(flat_ref, out_ref, m_ref):
    @pl.when(pl.program_id(0) == 0)
    def _build():
        col = jax.lax.broadcasted_iota(jnp.int32, (1, _D), 1)
        for r in range(_D):
            s = r * (r + 1) // 2
            vals = flat_ref[0, s:s + _D].reshape(1, _D)
            row = jnp.where(
                col < r, vals,
                jnp.where(col == r, jax.nn.softplus(vals), 0.0))
            m_ref[r:r + 1, :] = row

    out_ref[...] = jnp.broadcast_to(m_ref[...][None, :, :], (_TB, _D, _D))


def kernel(sigma2_schedule, t_schedule, flat_noise_sigma_chol):
    Tn = sigma2_schedule.shape[0]
    flat2 = jnp.concatenate(
        [flat_noise_sigma_chol,
         jnp.zeros((_D,), jnp.float32)]).reshape(1, _FLAT + _D)
    return pl.pallas_call(
        _body,
        grid=(Tn // _TB,),
        in_specs=[pl.BlockSpec((1, _FLAT + _D), lambda i: (0, 0))],
        out_specs=pl.BlockSpec((_TB, _D, _D), lambda i: (i, 0, 0)),
        out_shape=jax.ShapeDtypeStruct((Tn, _D, _D), jnp.float32),
        scratch_shapes=[pltpu.VMEM((_D, _D), jnp.float32)],
    )(flat2)


# speedup vs baseline: 8.4567x; 1.0616x over previous
import jax
import jax.numpy as jnp
from jax.experimental import pallas as pl
from jax.experimental.pallas import tpu as pltpu

_D = 128
_FLAT = _D * (_D + 1) // 2
_TB = 128


def _body(flat_ref, out_ref, m_ref):
    @pl.when(pl.program_id(0) == 0)
    def _build():
        col = jax.lax.broadcasted_iota(jnp.int32, (1, _D), 1)
        for r in range(_D):
            s = r * (r + 1) // 2
            vals = flat_ref[0, s:s + _D].reshape(1, _D)
            row = jnp.where(
                col < r, vals,
                jnp.where(col == r, jax.nn.softplus(vals), 0.0))
            m_ref[r:r + 1, :] = row

    out_ref[...] = jnp.broadcast_to(m_ref[...][None, :, :], (_TB, _D, _D))


def kernel(sigma2_schedule, t_schedule, flat_noise_sigma_chol):
    Tn = sigma2_schedule.shape[0]
    flat2 = jnp.concatenate(
        [flat_noise_sigma_chol,
         jnp.zeros((_D,), jnp.float32)]).reshape(1, _FLAT + _D)
    return pl.pallas_call(
        _body,
        grid=(Tn // _TB,),
        in_specs=[pl.BlockSpec((1, _FLAT + _D), lambda i: (0, 0))],
        out_specs=pl.BlockSpec((_TB, _D, _D), lambda i: (i, 0, 0)),
        out_shape=jax.ShapeDtypeStruct((Tn, _D, _D), jnp.float32),
        scratch_shapes=[pltpu.VMEM((_D, _D), jnp.float32)],
    )(flat2)


# speedup vs baseline: 8.5733x; 1.0138x over previous
import jax
import jax.numpy as jnp
from jax.experimental import pallas as pl
from jax.experimental.pallas import tpu as pltpu

_D = 128
_FLAT = _D * (_D + 1) // 2
_TB = 128
_NSEM = 4


def _body(flat_ref, out_ref, m_ref, buf_ref, sems):
    col = jax.lax.broadcasted_iota(jnp.int32, (1, _D), 1)
    for r in range(_D):
        s = r * (r + 1) // 2
        vals = flat_ref[0, s:s + _D].reshape(1, _D)
        row = jnp.where(
            col < r, vals,
            jnp.where(col == r, jax.nn.softplus(vals), 0.0))
        m_ref[r:r + 1, :] = row

    buf_ref[...] = jnp.broadcast_to(m_ref[...][None, :, :], (_TB, _D, _D))

    n = out_ref.shape[0] // _TB

    def issue(i, _):
        for k in range(_NSEM):
            pltpu.make_async_copy(
                buf_ref, out_ref.at[pl.ds((i * _NSEM + k) * _TB, _TB)],
                sems.at[k]).start()
        return 0

    jax.lax.fori_loop(0, n // _NSEM, issue, 0)

    def drain(i, _):
        for k in range(_NSEM):
            pltpu.make_async_copy(
                buf_ref, out_ref.at[pl.ds(0, _TB)], sems.at[k]).wait()
        return 0

    jax.lax.fori_loop(0, n // _NSEM, drain, 0)


def kernel(sigma2_schedule, t_schedule, flat_noise_sigma_chol):
    Tn = sigma2_schedule.shape[0]
    flat2 = jnp.concatenate(
        [flat_noise_sigma_chol,
         jnp.zeros((_D,), jnp.float32)]).reshape(1, _FLAT + _D)
    return pl.pallas_call(
        _body,
        in_specs=[pl.BlockSpec((1, _FLAT + _D), lambda: (0, 0))],
        out_specs=pl.BlockSpec(memory_space=pl.ANY),
        out_shape=jax.ShapeDtypeStruct((Tn, _D, _D), jnp.float32),
        scratch_shapes=[
            pltpu.VMEM((_D, _D), jnp.float32),
            pltpu.VMEM((_TB, _D, _D), jnp.float32),
            pltpu.SemaphoreType.DMA((_NSEM,)),
        ],
    )(flat2)
